# baseline (speedup 1.0000x reference)
import jax
import jax.numpy as jnp
from jax.experimental import pallas as pl
from jax.experimental.pallas import tpu as pltpu

_NC = 16


def _fused_kernel(x_ref, w_hbm, le_hbm, b_ref, o_ref,
                  wbf_ref, wtmp_ref, letmp_ref, bias_ref, wsem, lesem):
    nc = _NC
    ch = w_hbm.shape[0] // nc

    @pl.when(pl.program_id(0) == 0)
    def _prologue():
        def w_copy(c):
            return pltpu.make_async_copy(
                w_hbm.at[pl.ds(c * ch, ch), :], wtmp_ref.at[c % 2], wsem.at[c % 2])

        def le_copy(c):
            return pltpu.make_async_copy(
                le_hbm.at[pl.ds(c * ch, ch), :], letmp_ref.at[c % 2], lesem.at[c % 2])

        w_copy(0).start()
        for c in range(nc):
            if c + 1 < nc:
                w_copy(c + 1).start()
            w_copy(c).wait()
            wbf_ref[pl.ds(c * ch, ch), :] = wtmp_ref[c % 2].astype(jnp.bfloat16)
        le_copy(0).start()
        for c in range(nc):
            if c + 1 < nc:
                le_copy(c + 1).start()
            le_copy(c).wait()
            w32 = wbf_ref[pl.ds(c * ch, ch), :].astype(jnp.float32)
            bias_ref[0, pl.ds(c * ch, ch)] = (
                jnp.sum(w32 * letmp_ref[c % 2], axis=1)
                + b_ref[0, pl.ds(c * ch, ch)])

    x_bf = x_ref[...].astype(jnp.bfloat16)
    acc = jax.lax.dot_general(
        x_bf, wbf_ref[...],
        dimension_numbers=(((1,), (1,)), ((), ())),
        preferred_element_type=jnp.float32,
    )
    o_ref[...] = acc + bias_ref[...]


def kernel(bert_output, label_embed, W, b):
    B, H = bert_output.shape
    L = W.shape[0]
    bm = min(1024, B)
    nb = B // bm
    ch = L // _NC

    return pl.pallas_call(
        _fused_kernel,
        grid=(nb,),
        in_specs=[
            pl.BlockSpec((bm, H), lambda j: (j, 0)),
            pl.BlockSpec(memory_space=pl.ANY),
            pl.BlockSpec(memory_space=pl.ANY),
            pl.BlockSpec((1, L), lambda j: (0, 0)),
        ],
        out_specs=pl.BlockSpec((bm, L), lambda j: (j, 0)),
        out_shape=jax.ShapeDtypeStruct((B, L), jnp.float32),
        scratch_shapes=[
            pltpu.VMEM((L, H), jnp.bfloat16),
            pltpu.VMEM((2, ch, H), jnp.float32),
            pltpu.VMEM((2, ch, H), jnp.float32),
            pltpu.VMEM((1, L), jnp.float32),
            pltpu.SemaphoreType.DMA((2,)),
            pltpu.SemaphoreType.DMA((2,)),
        ],
        compiler_params=pltpu.CompilerParams(
            dimension_semantics=("arbitrary",),
        ),
    )(bert_output, W, label_embed, b.reshape(1, L))


# speedup vs baseline: 1.0536x; 1.0536x over previous
import jax
import jax.numpy as jnp
from jax.experimental import pallas as pl
from jax.experimental.pallas import tpu as pltpu

_NC = 16


def _fused_kernel(x_ref, w_hbm, le_hbm, b_ref, o_ref,
                  wbf_ref, wtmp_ref, letmp_ref, bias_ref, wsem, lesem):
    nc = _NC
    ch = w_hbm.shape[0] // nc

    @pl.when(pl.program_id(0) == 0)
    def _prologue():
        def w_copy(c):
            return pltpu.make_async_copy(
                w_hbm.at[pl.ds(c * ch, ch), :], wtmp_ref.at[c % 2], wsem.at[c % 2])

        def le_copy(c):
            return pltpu.make_async_copy(
                le_hbm.at[pl.ds(c * ch, ch), :], letmp_ref.at[c % 2], lesem.at[c % 2])

        w_copy(0).start()
        le_copy(0).start()
        for c in range(nc):
            if c + 1 < nc:
                w_copy(c + 1).start()
                le_copy(c + 1).start()
            w_copy(c).wait()
            w = wtmp_ref[c % 2]
            wbf_ref[pl.ds(c * ch, ch), :] = w.astype(jnp.bfloat16)
            le_copy(c).wait()
            bias_ref[0, pl.ds(c * ch, ch)] = (
                jnp.sum(w * letmp_ref[c % 2], axis=1) + b_ref[0, pl.ds(c * ch, ch)])

    x_bf = x_ref[...].astype(jnp.bfloat16)
    acc = jax.lax.dot_general(
        x_bf, wbf_ref[...],
        dimension_numbers=(((1,), (1,)), ((), ())),
        preferred_element_type=jnp.float32,
    )
    o_ref[...] = acc + bias_ref[...]


def kernel(bert_output, label_embed, W, b):
    B, H = bert_output.shape
    L = W.shape[0]
    bm = min(1024, B)
    nb = B // bm
    ch = L // _NC

    return pl.pallas_call(
        _fused_kernel,
        grid=(nb,),
        in_specs=[
            pl.BlockSpec((bm, H), lambda j: (j, 0)),
            pl.BlockSpec(memory_space=pl.ANY),
            pl.BlockSpec(memory_space=pl.ANY),
            pl.BlockSpec((1, L), lambda j: (0, 0)),
        ],
        out_specs=pl.BlockSpec((bm, L), lambda j: (j, 0)),
        out_shape=jax.ShapeDtypeStruct((B, L), jnp.float32),
        scratch_shapes=[
            pltpu.VMEM((L, H), jnp.bfloat16),
            pltpu.VMEM((2, ch, H), jnp.float32),
            pltpu.VMEM((2, ch, H), jnp.float32),
            pltpu.VMEM((1, L), jnp.float32),
            pltpu.SemaphoreType.DMA((2,)),
            pltpu.SemaphoreType.DMA((2,)),
        ],
        compiler_params=pltpu.CompilerParams(
            dimension_semantics=("arbitrary",),
        ),
    )(bert_output, W, label_embed, b.reshape(1, L))
